# baseline (speedup 1.0000x reference)
import functools

import numpy as np

import jax
import jax.numpy as jnp
from jax import lax
from jax.experimental import pallas as pl
from jax.experimental.pallas import tpu as pltpu
from jax.sharding import Mesh, PartitionSpec as P


_B, _C, _H, _W = 16, 64, 32, 32
_K, _D = 1024, 64
_HW = _H * _W


_RB = 2
_R = _RB * _HW


def _vq_kernel(x_ref, w_ref, w2_ref, oh_ref, zq_ref):
    x = x_ref[...]
    z = jnp.concatenate([x[i].T for i in range(_RB)], axis=0)
    w = w_ref[...]

    z2 = jnp.sum(z * z, axis=1, keepdims=True)
    w2 = w2_ref[0]
    mm2 = lax.dot_general(z, w + w, (((1,), (1,)), ((), ())),
                          preferred_element_type=jnp.float32)
    d = (z2 + w2[None, :]) - mm2

    iota_i = jnp.broadcast_to(
        lax.broadcasted_iota(jnp.int32, (1, _K), 1) + (1 << 23), (_R, _K))
    dmin = jnp.min(d, axis=1, keepdims=True)
    key = lax.bitcast_convert_type(
        lax.bitcast_convert_type(d - dmin, jnp.int32) + iota_i,
        jnp.float32)
    kmin = jnp.min(key, axis=1, keepdims=True)
    oh = (key == kmin).astype(jnp.float32)
    oh_ref[...] = oh

    zq_t = lax.dot_general(w, oh, (((0,), (1,)), ((), ())),
                           preferred_element_type=jnp.float32)
    for i in range(_RB):
        zq_ref[i] = zq_t[:, i * _HW:(i + 1) * _HW]


def _vq_shard(Xr, W, w2):
    nb = Xr.shape[0]
    return pl.pallas_call(
        _vq_kernel,
        grid=(nb // _RB,),
        in_specs=[
            pl.BlockSpec((_RB, _C, _HW), lambda b: (b, 0, 0)),
            pl.BlockSpec((_K, _D), lambda b: (0, 0)),
            pl.BlockSpec((1, _K), lambda b: (0, 0)),
        ],
        out_specs=[
            pl.BlockSpec((_R, _K), lambda b: (b, 0)),
            pl.BlockSpec((_RB, _C, _HW), lambda b: (b, 0, 0)),
        ],
        out_shape=[
            jax.ShapeDtypeStruct((nb * _HW, _K), jnp.float32),
            jax.ShapeDtypeStruct((nb, _C, _HW), jnp.float32),
        ],
        compiler_params=pltpu.CompilerParams(
            dimension_semantics=("arbitrary",),
        ),
    )(Xr, W, w2)


@functools.partial(jax.jit, static_argnums=())
def kernel(X, W):
    Xr = X.reshape(_B, _C, _HW)
    w2 = jnp.sum(W ** 2, axis=1).reshape(1, _K)
    devs = jax.devices()
    ndev = 1
    if ndev > 1:
        mesh = Mesh(np.array(devs[:ndev]), ("b",))
        f = jax.shard_map(
            _vq_shard, mesh=mesh,
            in_specs=(P("b", None, None), P(None, None), P(None, None)),
            out_specs=(P("b", None), P("b", None, None)),
            check_vma=False,
        )
    else:
        f = _vq_shard
    oh, zq = f(Xr, W, w2)
    return (zq.reshape(_B, _C, _H, _W), oh)


# speedup vs baseline: 1.0057x; 1.0057x over previous
import functools

import numpy as np

import jax
import jax.numpy as jnp
from jax import lax
from jax.experimental import pallas as pl
from jax.experimental.pallas import tpu as pltpu
from jax.sharding import Mesh, PartitionSpec as P


_B, _C, _H, _W = 16, 64, 32, 32
_K, _D = 1024, 64
_HW = _H * _W


_RB = 2
_R = _RB * _HW


def _vq_kernel(x_ref, w_ref, w2_ref, oh_ref, zq_ref):
    x = x_ref[...]
    z = jnp.concatenate([x[i].T for i in range(_RB)], axis=0)
    w = w_ref[...]

    z2 = jnp.sum(z * z, axis=1, keepdims=True)
    w2 = w2_ref[0]
    mm2 = lax.dot_general(z, w + w, (((1,), (1,)), ((), ())),
                          preferred_element_type=jnp.float32)
    d = (z2 + w2[None, :]) - mm2

    iota_i = jnp.broadcast_to(
        lax.broadcasted_iota(jnp.int32, (1, _K), 1) + (1 << 23), (_R, _K))
    dmin = jnp.min(d, axis=1, keepdims=True)
    key = lax.bitcast_convert_type(
        lax.bitcast_convert_type(d - dmin, jnp.int32) + iota_i,
        jnp.float32)
    kmin = jnp.min(key, axis=1, keepdims=True)
    oh = (key == kmin).astype(jnp.float32)
    oh_ref[...] = oh

    zq_t = lax.dot_general(w.astype(jnp.bfloat16), oh.astype(jnp.bfloat16),
                           (((0,), (1,)), ((), ())),
                           preferred_element_type=jnp.float32)
    for i in range(_RB):
        zq_ref[i] = zq_t[:, i * _HW:(i + 1) * _HW]


def _vq_shard(Xr, W, w2):
    nb = Xr.shape[0]
    return pl.pallas_call(
        _vq_kernel,
        grid=(nb // _RB,),
        in_specs=[
            pl.BlockSpec((_RB, _C, _HW), lambda b: (b, 0, 0)),
            pl.BlockSpec((_K, _D), lambda b: (0, 0)),
            pl.BlockSpec((1, _K), lambda b: (0, 0)),
        ],
        out_specs=[
            pl.BlockSpec((_R, _K), lambda b: (b, 0)),
            pl.BlockSpec((_RB, _C, _HW), lambda b: (b, 0, 0)),
        ],
        out_shape=[
            jax.ShapeDtypeStruct((nb * _HW, _K), jnp.float32),
            jax.ShapeDtypeStruct((nb, _C, _HW), jnp.float32),
        ],
        compiler_params=pltpu.CompilerParams(
            dimension_semantics=("arbitrary",),
        ),
    )(Xr, W, w2)


@functools.partial(jax.jit, static_argnums=())
def kernel(X, W):
    Xr = X.reshape(_B, _C, _HW)
    w2 = jnp.sum(W ** 2, axis=1).reshape(1, _K)
    devs = jax.devices()
    ndev = 1
    if ndev > 1:
        mesh = Mesh(np.array(devs[:ndev]), ("b",))
        f = jax.shard_map(
            _vq_shard, mesh=mesh,
            in_specs=(P("b", None, None), P(None, None), P(None, None)),
            out_specs=(P("b", None), P("b", None, None)),
            check_vma=False,
        )
    else:
        f = _vq_shard
    oh, zq = f(Xr, W, w2)
    return (zq.reshape(_B, _C, _H, _W), oh)


# speedup vs baseline: 1.0073x; 1.0016x over previous
import functools

import jax
import jax.numpy as jnp
from jax import lax
from jax.experimental import pallas as pl
from jax.experimental.pallas import tpu as pltpu


_B, _C, _H, _W = 16, 64, 32, 32
_K, _D = 1024, 64
_HW = _H * _W

_RB = 2
_R = _RB * _HW


def _vq_kernel(x_ref, w_ref, w2_ref, oh_ref, zq_ref):
    x = x_ref[...]
    z = jnp.concatenate([x[i].T for i in range(_RB)], axis=0)
    w = w_ref[...]

    z2 = jnp.sum(z * z, axis=1, keepdims=True)
    w2 = w2_ref[0]
    mm2 = lax.dot_general(z, w + w, (((1,), (1,)), ((), ())),
                          preferred_element_type=jnp.float32)
    d = (z2 + w2[None, :]) - mm2

    iota_i = jnp.broadcast_to(
        lax.broadcasted_iota(jnp.int32, (1, _K), 1) + (1 << 23), (_R, _K))
    dmin = jnp.min(d, axis=1, keepdims=True)
    key = lax.bitcast_convert_type(
        lax.bitcast_convert_type(d - dmin, jnp.int32) + iota_i,
        jnp.float32)
    kmin = jnp.min(key, axis=1, keepdims=True)
    oh = (key == kmin).astype(jnp.float32)
    oh_ref[...] = oh

    zq_t = lax.dot_general(w.astype(jnp.bfloat16), oh.astype(jnp.bfloat16),
                           (((0,), (1,)), ((), ())),
                           preferred_element_type=jnp.float32)
    for i in range(_RB):
        zq_ref[i] = zq_t[:, i * _HW:(i + 1) * _HW]


@functools.partial(jax.jit, static_argnums=())
def kernel(X, W):
    Xr = X.reshape(_B, _C, _HW)
    w2 = jnp.sum(W ** 2, axis=1).reshape(1, _K)
    oh, zq = pl.pallas_call(
        _vq_kernel,
        grid=(_B // _RB,),
        in_specs=[
            pl.BlockSpec((_RB, _C, _HW), lambda b: (b, 0, 0)),
            pl.BlockSpec((_K, _D), lambda b: (0, 0)),
            pl.BlockSpec((1, _K), lambda b: (0, 0)),
        ],
        out_specs=[
            pl.BlockSpec((_R, _K), lambda b: (b, 0)),
            pl.BlockSpec((_RB, _C, _HW), lambda b: (b, 0, 0)),
        ],
        out_shape=[
            jax.ShapeDtypeStruct((_B * _HW, _K), jnp.float32),
            jax.ShapeDtypeStruct((_B, _C, _HW), jnp.float32),
        ],
        compiler_params=pltpu.CompilerParams(
            dimension_semantics=("arbitrary",),
        ),
    )(Xr, W, w2)
    return (zq.reshape(_B, _C, _H, _W), oh)
